# baseline (speedup 1.0000x reference)
import jax
import jax.numpy as jnp
import numpy as np
from jax import lax
from jax.experimental import pallas as pl
from jax.experimental.pallas import tpu as pltpu
from jax.experimental.pallas import tpu_sc as plsc

N = 10000
E = 320000
D = 128
OC = 64
MIN_NORM = 1e-15
MAXNORM = 1.0 - 4e-3

R_BLK = 1000
C_CHUNK = 1600
UNROLL = 4
NCH = E // C_CHUNK


_F32_ONE_EPS = np.float32(1.0 - 1e-7)
_F32_MIN_NORM = np.float32(1e-15)


def _artanh(v):
    v = jnp.minimum(jnp.maximum(v, -_F32_ONE_EPS), _F32_ONE_EPS)
    return 0.5 * (jnp.log1p(v) - jnp.log1p(-v))


def _norm(v):
    return jnp.maximum(jnp.sqrt(jnp.sum(v * v, -1, keepdims=True)), _F32_MIN_NORM)


def _proj(v):
    n = _norm(v)
    return jnp.where(n > MAXNORM, v / n * MAXNORM, v)


def _expmap0(u):
    n = _norm(u)
    return jnp.tanh(n) * u / n


def _logmap0(p):
    n = _norm(p)
    return _artanh(n) * p / n


def _pre_body(x_ref, w_ref, b_ref, atti_ref, attj_ref, xt0_ref, scal_ref):
    W = w_ref[...]
    hw = _proj(_expmap0(W))
    x = x_ref[...]
    x_norm = _norm(x)
    mx = lax.dot_general(x, hw, (((1,), (1,)), ((), ())),
                         precision=lax.Precision.HIGHEST,
                         preferred_element_type=jnp.float32)
    mx_norm = _norm(mx)
    res = jnp.tanh(mx_norm / x_norm * _artanh(x_norm)) * mx / mx_norm
    zero_mask = jnp.max(jnp.abs(mx), -1, keepdims=True) == 0.0
    res = jnp.where(zero_mask, np.float32(0.0), res)
    res = _proj(res)
    hb = _proj(_expmap0(b_ref[...]))
    x2 = jnp.sum(res * res, -1, keepdims=True)
    y2 = jnp.sum(hb * hb, -1, keepdims=True)
    xy = jnp.sum(res * hb, -1, keepdims=True)
    num = (1.0 + 2.0 * xy + y2) * res + (1.0 - x2) * hb
    den = 1.0 + 2.0 * xy + x2 * y2
    h = _proj(num / jnp.maximum(den, _F32_MIN_NORM))
    xt0 = _logmap0(h)
    xt0_ref[...] = xt0

    pi = xt0 * atti_ref[...]
    pj = xt0 * attj_ref[...]
    ai0 = jnp.sum(pi[:, :OC], -1, keepdims=True)
    ai1 = jnp.sum(pi[:, OC:], -1, keepdims=True)
    aj0 = jnp.sum(pj[:, :OC], -1, keepdims=True)
    aj1 = jnp.sum(pj[:, OC:], -1, keepdims=True)
    s0 = ai0 + aj0
    s1 = ai1 + aj1
    exsl0 = jnp.exp(jnp.maximum(s0, 0.2 * s0))
    exsl1 = jnp.exp(jnp.maximum(s1, 0.2 * s1))
    z = jnp.zeros_like(ai0)
    scal_ref[...] = jnp.concatenate(
        [ai0, ai1, aj0, aj1, exsl0, exsl1, z, z], axis=1)


def _sc_agg(xt0t, srcv, dstv, out,
            feat0, feat1, feat2, feat3, tab_ai, tab_aj,
            acc0, acc1, acc2, acc3, dacc,
            srcA, srcB, dstA, dstB,
            sem_sa, sem_sb, sem_da, sem_db):
    i32 = jnp.int32
    cid = i32(lax.axis_index("c"))
    sid = i32(lax.axis_index("s"))
    wid = sid * 2 + cid
    base = wid * 4
    head = wid // 16

    pltpu.sync_copy(xt0t.at[base], feat0)
    pltpu.sync_copy(xt0t.at[base + 1], feat1)
    pltpu.sync_copy(xt0t.at[base + 2], feat2)
    pltpu.sync_copy(xt0t.at[base + 3], feat3)
    pltpu.sync_copy(xt0t.at[128 + head], tab_ai)
    pltpu.sync_copy(xt0t.at[130 + head], tab_aj)

    zeros = jnp.zeros((16,), jnp.float32)

    def zbody(i, carry):
        sl = pl.ds(i * 16, 16)
        acc0[sl] = zeros
        acc1[sl] = zeros
        acc2[sl] = zeros
        acc3[sl] = zeros
        dacc[sl] = zeros
        return carry

    lax.fori_loop(i32(0), i32(N // 16), zbody, i32(0))

    pltpu.async_copy(srcv.at[pl.ds(0, C_CHUNK)], srcA, sem_sa)
    pltpu.async_copy(dstv.at[pl.ds(0, C_CHUNK)], dstA, sem_da)
    pltpu.async_copy(srcv.at[pl.ds(C_CHUNK, C_CHUNK)], srcB, sem_sb)
    pltpu.async_copy(dstv.at[pl.ds(C_CHUNK, C_CHUNK)], dstB, sem_db)

    is_den = jnp.logical_or(wid == 0, wid == 16)

    def chunk_body(k, carry):
        for p, (sbuf, dbuf, ssem, dsem) in enumerate(
                ((srcA, dstA, sem_sa, sem_da), (srcB, dstB, sem_sb, sem_db))):
            g = 2 * k + p
            pltpu.make_async_copy(
                srcv.at[pl.ds(g * C_CHUNK, C_CHUNK)], sbuf, ssem).wait()
            pltpu.make_async_copy(
                dstv.at[pl.ds(g * C_CHUNK, C_CHUNK)], dbuf, dsem).wait()

            def gbody(j, c2):
                for u in range(UNROLL):
                    sl = pl.ds((j * UNROLL + u) * 16, 16)
                    s = sbuf[sl]
                    d = dbuf[sl]
                    al = (plsc.load_gather(tab_ai, [s])
                          + plsc.load_gather(tab_aj, [d]))
                    al = jnp.maximum(al, 0.2 * al)
                    w = jnp.exp(al)
                    w = jnp.where(s == d, jnp.zeros_like(w), w)
                    v0 = plsc.load_gather(feat0, [d])
                    plsc.addupdate_scatter(acc0, [s], v0 * w)
                    v1 = plsc.load_gather(feat1, [d])
                    plsc.addupdate_scatter(acc1, [s], v1 * w)
                    v2 = plsc.load_gather(feat2, [d])
                    plsc.addupdate_scatter(acc2, [s], v2 * w)
                    v3 = plsc.load_gather(feat3, [d])
                    plsc.addupdate_scatter(acc3, [s], v3 * w)

                    @pl.when(is_den)
                    def _():
                        plsc.addupdate_scatter(dacc, [s], w)

                return c2

            lax.fori_loop(i32(0), i32(C_CHUNK // (16 * UNROLL)), gbody, i32(0))

            gn = g + 2

            @pl.when(gn < NCH)
            def _():
                pltpu.async_copy(
                    srcv.at[pl.ds(gn * C_CHUNK, C_CHUNK)], sbuf, ssem)
                pltpu.async_copy(
                    dstv.at[pl.ds(gn * C_CHUNK, C_CHUNK)], dbuf, dsem)

        return carry

    lax.fori_loop(i32(0), i32(NCH // 2), chunk_body, i32(0))

    pltpu.sync_copy(acc0, out.at[base])
    pltpu.sync_copy(acc1, out.at[base + 1])
    pltpu.sync_copy(acc2, out.at[base + 2])
    pltpu.sync_copy(acc3, out.at[base + 3])

    @pl.when(wid == 0)
    def _():
        pltpu.sync_copy(dacc, out.at[i32(128)])

    @pl.when(wid == 16)
    def _():
        pltpu.sync_copy(dacc, out.at[i32(129)])


def _post_body(xt0_ref, scal_ref, acc_ref, out_ref):
    xt0 = xt0_ref[...]
    scal = scal_ref[...]
    acc = acc_ref[...]
    ex0 = scal[:, 4:5]
    ex1 = scal[:, 5:6]
    num0 = acc[:, :OC] + ex0 * xt0[:, :OC]
    num1 = acc[:, OC:D] + ex1 * xt0[:, OC:]
    den0 = acc[:, D:D + 1] + ex0
    den1 = acc[:, D + 1:D + 2] + ex1
    support = 0.5 * (num0 / den0 + num1 / den1)
    u = _proj(_expmap0(support))
    xt = _logmap0(u)
    xt = jnp.where(xt >= 0, xt, 0.01 * xt)
    out_ref[...] = _proj(_expmap0(xt))


def _build_calls():
    pre = pl.pallas_call(
        _pre_body,
        grid=(N // R_BLK,),
        in_specs=[
            pl.BlockSpec((R_BLK, D), lambda i: (i, jnp.int32(0))),
            pl.BlockSpec((D, D), lambda i: (jnp.int32(0), jnp.int32(0))),
            pl.BlockSpec((1, D), lambda i: (jnp.int32(0), jnp.int32(0))),
            pl.BlockSpec((1, D), lambda i: (jnp.int32(0), jnp.int32(0))),
            pl.BlockSpec((1, D), lambda i: (jnp.int32(0), jnp.int32(0))),
        ],
        out_specs=[
            pl.BlockSpec((R_BLK, D), lambda i: (i, jnp.int32(0))),
            pl.BlockSpec((R_BLK, 8), lambda i: (i, jnp.int32(0))),
        ],
        out_shape=[
            jax.ShapeDtypeStruct((N, D), jnp.float32),
            jax.ShapeDtypeStruct((N, 8), jnp.float32),
        ],
    )

    mesh = plsc.VectorSubcoreMesh(core_axis_name="c", subcore_axis_name="s")
    agg = pl.kernel(
        _sc_agg,
        out_type=jax.ShapeDtypeStruct((130, N), jnp.float32),
        mesh=mesh,
        compiler_params=pltpu.CompilerParams(needs_layout_passes=False),
        scratch_types=[
            pltpu.VMEM((N,), jnp.float32),
            pltpu.VMEM((N,), jnp.float32),
            pltpu.VMEM((N,), jnp.float32),
            pltpu.VMEM((N,), jnp.float32),
            pltpu.VMEM((N,), jnp.float32),
            pltpu.VMEM((N,), jnp.float32),
            pltpu.VMEM((N,), jnp.float32),
            pltpu.VMEM((N,), jnp.float32),
            pltpu.VMEM((N,), jnp.float32),
            pltpu.VMEM((N,), jnp.float32),
            pltpu.VMEM((N,), jnp.float32),
            pltpu.VMEM((C_CHUNK,), jnp.int32),
            pltpu.VMEM((C_CHUNK,), jnp.int32),
            pltpu.VMEM((C_CHUNK,), jnp.int32),
            pltpu.VMEM((C_CHUNK,), jnp.int32),
            pltpu.SemaphoreType.DMA,
            pltpu.SemaphoreType.DMA,
            pltpu.SemaphoreType.DMA,
            pltpu.SemaphoreType.DMA,
        ],
    )

    post = pl.pallas_call(
        _post_body,
        grid=(N // R_BLK,),
        in_specs=[
            pl.BlockSpec((R_BLK, D), lambda i: (i, jnp.int32(0))),
            pl.BlockSpec((R_BLK, 8), lambda i: (i, jnp.int32(0))),
            pl.BlockSpec((R_BLK, 130), lambda i: (i, jnp.int32(0))),
        ],
        out_specs=pl.BlockSpec((R_BLK, OC), lambda i: (i, jnp.int32(0))),
        out_shape=jax.ShapeDtypeStruct((N, OC), jnp.float32),
    )
    return pre, agg, post


_PRE, _AGG, _POST = _build_calls()


@jax.jit
def _run(x, src, dst, W, b2, atti, attj):
    xt0, scal = _PRE(x, W, b2, atti, attj)
    xt0t = jnp.concatenate([xt0, scal[:, :4]], axis=1).T
    acc = _AGG(xt0t, src, dst)
    accT = acc.T
    return _POST(xt0, scal, accT)


def kernel(x, edge_index, W, b, att_i, att_j):
    x = x.astype(jnp.float32)
    src = edge_index[0].astype(jnp.int32)
    dst = edge_index[1].astype(jnp.int32)
    W = W.astype(jnp.float32)
    b2 = b.reshape(1, D).astype(jnp.float32)
    atti = att_i.reshape(1, D).astype(jnp.float32)
    attj = att_j.reshape(1, D).astype(jnp.float32)
    return _run(x, src, dst, W, b2, atti, attj)


# speedup vs baseline: 2.6080x; 2.6080x over previous
import jax
import jax.numpy as jnp
import numpy as np
from jax import lax
from jax.experimental import pallas as pl
from jax.experimental.pallas import tpu as pltpu
from jax.experimental.pallas import tpu_sc as plsc

N = 10000
E = 320000
D = 128
OC = 64
MIN_NORM = 1e-15
MAXNORM = 1.0 - 4e-3

R_BLK = 1000
C_CHUNK = 1600
UNROLL = 4
NCH = E // C_CHUNK


_F32_ONE_EPS = np.float32(1.0 - 1e-7)
_F32_MIN_NORM = np.float32(1e-15)


def _artanh(v):
    v = jnp.minimum(jnp.maximum(v, -_F32_ONE_EPS), _F32_ONE_EPS)
    return 0.5 * (jnp.log1p(v) - jnp.log1p(-v))


def _norm(v):
    return jnp.maximum(jnp.sqrt(jnp.sum(v * v, -1, keepdims=True)), _F32_MIN_NORM)


def _proj(v):
    n = _norm(v)
    return jnp.where(n > MAXNORM, v / n * MAXNORM, v)


def _expmap0(u):
    n = _norm(u)
    return jnp.tanh(n) * u / n


def _logmap0(p):
    n = _norm(p)
    return _artanh(n) * p / n


def _pre_body(x_ref, w_ref, b_ref, atti_ref, attj_ref, xt0_ref, scal_ref):
    W = w_ref[...]
    hw = _proj(_expmap0(W))
    x = x_ref[...]
    x_norm = _norm(x)
    mx = lax.dot_general(x, hw, (((1,), (1,)), ((), ())),
                         precision=lax.Precision.HIGHEST,
                         preferred_element_type=jnp.float32)
    mx_norm = _norm(mx)
    res = jnp.tanh(mx_norm / x_norm * _artanh(x_norm)) * mx / mx_norm
    zero_mask = jnp.max(jnp.abs(mx), -1, keepdims=True) == 0.0
    res = jnp.where(zero_mask, np.float32(0.0), res)
    res = _proj(res)
    hb = _proj(_expmap0(b_ref[...]))
    x2 = jnp.sum(res * res, -1, keepdims=True)
    y2 = jnp.sum(hb * hb, -1, keepdims=True)
    xy = jnp.sum(res * hb, -1, keepdims=True)
    num = (1.0 + 2.0 * xy + y2) * res + (1.0 - x2) * hb
    den = 1.0 + 2.0 * xy + x2 * y2
    h = _proj(num / jnp.maximum(den, _F32_MIN_NORM))
    xt0 = _logmap0(h)
    xt0_ref[...] = xt0

    pi = xt0 * atti_ref[...]
    pj = xt0 * attj_ref[...]
    ai0 = jnp.sum(pi[:, :OC], -1, keepdims=True)
    ai1 = jnp.sum(pi[:, OC:], -1, keepdims=True)
    aj0 = jnp.sum(pj[:, :OC], -1, keepdims=True)
    aj1 = jnp.sum(pj[:, OC:], -1, keepdims=True)
    s0 = ai0 + aj0
    s1 = ai1 + aj1
    exsl0 = jnp.exp(jnp.maximum(s0, 0.2 * s0))
    exsl1 = jnp.exp(jnp.maximum(s1, 0.2 * s1))
    z = jnp.zeros_like(ai0)
    scal_ref[...] = jnp.concatenate(
        [ai0, ai1, aj0, aj1, exsl0, exsl1, z, z], axis=1)


def _sc_agg(xt0t, srcv, dstv, out,
            feat0, feat1, feat2, feat3, tab_ai, tab_aj,
            acc0, acc1, acc2, acc3, dacc,
            srcA, srcB, dstA, dstB,
            sem_sa, sem_sb, sem_da, sem_db):
    i32 = jnp.int32
    cid = i32(lax.axis_index("c"))
    sid = i32(lax.axis_index("s"))
    wid = sid * 2 + cid
    base = wid * 4
    head = wid // 16

    pltpu.sync_copy(xt0t.at[base], feat0)
    pltpu.sync_copy(xt0t.at[base + 1], feat1)
    pltpu.sync_copy(xt0t.at[base + 2], feat2)
    pltpu.sync_copy(xt0t.at[base + 3], feat3)
    pltpu.sync_copy(xt0t.at[128 + head], tab_ai)
    pltpu.sync_copy(xt0t.at[130 + head], tab_aj)

    zeros = jnp.zeros((16,), jnp.float32)

    def zbody(i, carry):
        sl = pl.ds(i * 16, 16)
        acc0[sl] = zeros
        acc1[sl] = zeros
        acc2[sl] = zeros
        acc3[sl] = zeros
        dacc[sl] = zeros
        return carry

    lax.fori_loop(i32(0), i32(N // 16), zbody, i32(0))

    pltpu.async_copy(srcv.at[pl.ds(0, C_CHUNK)], srcA, sem_sa)
    pltpu.async_copy(dstv.at[pl.ds(0, C_CHUNK)], dstA, sem_da)
    pltpu.async_copy(srcv.at[pl.ds(C_CHUNK, C_CHUNK)], srcB, sem_sb)
    pltpu.async_copy(dstv.at[pl.ds(C_CHUNK, C_CHUNK)], dstB, sem_db)

    is_den = jnp.logical_or(wid == 0, wid == 16)

    def chunk_body(k, carry):
        for p, (sbuf, dbuf, ssem, dsem) in enumerate(
                ((srcA, dstA, sem_sa, sem_da), (srcB, dstB, sem_sb, sem_db))):
            g = 2 * k + p
            pltpu.make_async_copy(
                srcv.at[pl.ds(g * C_CHUNK, C_CHUNK)], sbuf, ssem).wait()
            pltpu.make_async_copy(
                dstv.at[pl.ds(g * C_CHUNK, C_CHUNK)], dbuf, dsem).wait()

            @plsc.parallel_loop(np.int32(0), np.int32(C_CHUNK // 16), np.int32(1), unroll=UNROLL)
            def gbody(j):
                sl = pl.ds(j * 16, 16)
                s = sbuf[sl]
                d = dbuf[sl]
                al = (plsc.load_gather(tab_ai, [s])
                      + plsc.load_gather(tab_aj, [d]))
                al = jnp.maximum(al, 0.2 * al)
                w = jnp.exp(al)
                w = jnp.where(s == d, jnp.zeros_like(w), w)
                v0 = plsc.load_gather(feat0, [d])
                plsc.addupdate_scatter(acc0, [s], v0 * w)
                v1 = plsc.load_gather(feat1, [d])
                plsc.addupdate_scatter(acc1, [s], v1 * w)
                v2 = plsc.load_gather(feat2, [d])
                plsc.addupdate_scatter(acc2, [s], v2 * w)
                v3 = plsc.load_gather(feat3, [d])
                plsc.addupdate_scatter(acc3, [s], v3 * w)

                @pl.when(is_den)
                def _():
                    plsc.addupdate_scatter(dacc, [s], w)

            gn = g + 2

            @pl.when(gn < NCH)
            def _():
                pltpu.async_copy(
                    srcv.at[pl.ds(gn * C_CHUNK, C_CHUNK)], sbuf, ssem)
                pltpu.async_copy(
                    dstv.at[pl.ds(gn * C_CHUNK, C_CHUNK)], dbuf, dsem)

        return carry

    lax.fori_loop(i32(0), i32(NCH // 2), chunk_body, i32(0))

    pltpu.sync_copy(acc0, out.at[base])
    pltpu.sync_copy(acc1, out.at[base + 1])
    pltpu.sync_copy(acc2, out.at[base + 2])
    pltpu.sync_copy(acc3, out.at[base + 3])

    @pl.when(wid == 0)
    def _():
        pltpu.sync_copy(dacc, out.at[i32(128)])

    @pl.when(wid == 16)
    def _():
        pltpu.sync_copy(dacc, out.at[i32(129)])


def _post_body(xt0_ref, scal_ref, acc_ref, out_ref):
    xt0 = xt0_ref[...]
    scal = scal_ref[...]
    acc = acc_ref[...]
    ex0 = scal[:, 4:5]
    ex1 = scal[:, 5:6]
    num0 = acc[:, :OC] + ex0 * xt0[:, :OC]
    num1 = acc[:, OC:D] + ex1 * xt0[:, OC:]
    den0 = acc[:, D:D + 1] + ex0
    den1 = acc[:, D + 1:D + 2] + ex1
    support = 0.5 * (num0 / den0 + num1 / den1)
    u = _proj(_expmap0(support))
    xt = _logmap0(u)
    xt = jnp.where(xt >= 0, xt, 0.01 * xt)
    out_ref[...] = _proj(_expmap0(xt))


def _build_calls():
    pre = pl.pallas_call(
        _pre_body,
        grid=(N // R_BLK,),
        in_specs=[
            pl.BlockSpec((R_BLK, D), lambda i: (i, jnp.int32(0))),
            pl.BlockSpec((D, D), lambda i: (jnp.int32(0), jnp.int32(0))),
            pl.BlockSpec((1, D), lambda i: (jnp.int32(0), jnp.int32(0))),
            pl.BlockSpec((1, D), lambda i: (jnp.int32(0), jnp.int32(0))),
            pl.BlockSpec((1, D), lambda i: (jnp.int32(0), jnp.int32(0))),
        ],
        out_specs=[
            pl.BlockSpec((R_BLK, D), lambda i: (i, jnp.int32(0))),
            pl.BlockSpec((R_BLK, 8), lambda i: (i, jnp.int32(0))),
        ],
        out_shape=[
            jax.ShapeDtypeStruct((N, D), jnp.float32),
            jax.ShapeDtypeStruct((N, 8), jnp.float32),
        ],
    )

    mesh = plsc.VectorSubcoreMesh(core_axis_name="c", subcore_axis_name="s")
    agg = pl.kernel(
        _sc_agg,
        out_type=jax.ShapeDtypeStruct((130, N), jnp.float32),
        mesh=mesh,
        compiler_params=pltpu.CompilerParams(needs_layout_passes=False),
        scratch_types=[
            pltpu.VMEM((N,), jnp.float32),
            pltpu.VMEM((N,), jnp.float32),
            pltpu.VMEM((N,), jnp.float32),
            pltpu.VMEM((N,), jnp.float32),
            pltpu.VMEM((N,), jnp.float32),
            pltpu.VMEM((N,), jnp.float32),
            pltpu.VMEM((N,), jnp.float32),
            pltpu.VMEM((N,), jnp.float32),
            pltpu.VMEM((N,), jnp.float32),
            pltpu.VMEM((N,), jnp.float32),
            pltpu.VMEM((N,), jnp.float32),
            pltpu.VMEM((C_CHUNK,), jnp.int32),
            pltpu.VMEM((C_CHUNK,), jnp.int32),
            pltpu.VMEM((C_CHUNK,), jnp.int32),
            pltpu.VMEM((C_CHUNK,), jnp.int32),
            pltpu.SemaphoreType.DMA,
            pltpu.SemaphoreType.DMA,
            pltpu.SemaphoreType.DMA,
            pltpu.SemaphoreType.DMA,
        ],
    )

    post = pl.pallas_call(
        _post_body,
        grid=(N // R_BLK,),
        in_specs=[
            pl.BlockSpec((R_BLK, D), lambda i: (i, jnp.int32(0))),
            pl.BlockSpec((R_BLK, 8), lambda i: (i, jnp.int32(0))),
            pl.BlockSpec((R_BLK, 130), lambda i: (i, jnp.int32(0))),
        ],
        out_specs=pl.BlockSpec((R_BLK, OC), lambda i: (i, jnp.int32(0))),
        out_shape=jax.ShapeDtypeStruct((N, OC), jnp.float32),
    )
    return pre, agg, post


_PRE, _AGG, _POST = _build_calls()


@jax.jit
def _run(x, src, dst, W, b2, atti, attj):
    xt0, scal = _PRE(x, W, b2, atti, attj)
    xt0t = jnp.concatenate([xt0, scal[:, :4]], axis=1).T
    acc = _AGG(xt0t, src, dst)
    accT = acc.T
    return _POST(xt0, scal, accT)


def kernel(x, edge_index, W, b, att_i, att_j):
    x = x.astype(jnp.float32)
    src = edge_index[0].astype(jnp.int32)
    dst = edge_index[1].astype(jnp.int32)
    W = W.astype(jnp.float32)
    b2 = b.reshape(1, D).astype(jnp.float32)
    atti = att_i.reshape(1, D).astype(jnp.float32)
    attj = att_j.reshape(1, D).astype(jnp.float32)
    return _run(x, src, dst, W, b2, atti, attj)
